# baseline (speedup 1.0000x reference)
import functools

import jax
import jax.numpy as jnp
from jax import lax
from jax.experimental import pallas as pl
from jax.experimental.pallas import tpu as pltpu
from jax.experimental.pallas import tpu_sc as plsc

_K = 1024
_D = 64
_TB = 4096

_NW = 32
_CH = 64


def _vq_block(xt_ref, wt_ref, loss_ref, qt_ref, perp_ref, idx_ref,
              loss_acc, hist_acc, wsq_acc, *, nblocks, n_tokens):
    i = pl.program_id(0)
    wt = wt_ref[...]

    @pl.when(i == 0)
    def _init():
        loss_acc[0, 0] = 0.0
        hist_acc[...] = jnp.zeros_like(hist_acc)
        w = jnp.transpose(wt, (1, 0))
        wsq_acc[...] = jnp.sum(w * w, axis=1)[None, :]

    xc = xt_ref[...]
    nchunks, _, s = xc.shape
    x = jnp.transpose(xc, (0, 2, 1)).reshape(-1, _D)
    x_sq = jnp.sum(x * x, axis=1, keepdims=True)
    w_sq = wsq_acc[...]
    xw = jax.lax.dot_general(x, wt, (((1,), (0,)), ((), ())))
    dist = x_sq + w_sq - 2.0 * xw
    idx = jnp.argmin(dist, axis=1)
    idx_ref[...] = idx.reshape(1, 1, -1)
    iota = jax.lax.broadcasted_iota(jnp.int32, (x.shape[0], _K), 1)
    one_hot = (iota == idx[:, None]).astype(jnp.float32)
    loss_new = loss_acc[0, 0]
    for c in range(nchunks):
        oh_c = one_hot[c * s:(c + 1) * s, :]
        q_tc = jax.lax.dot_general(wt, oh_c, (((1,), (1,)), ((), ())))
        qt_ref[c] = q_tc
        d = q_tc - xc[c]
        loss_new = loss_new + jnp.sum(d * d)
    loss_acc[0, 0] = loss_new
    ones_row = jnp.ones((1, x.shape[0]), jnp.float32)
    hist_acc[...] += jax.lax.dot_general(
        ones_row, one_hot, (((1,), (0,)), ((), ())))

    @pl.when(i == nblocks - 1)
    def _fin():
        m = loss_acc[0, 0] / (n_tokens * _D)
        loss_ref[...] = jnp.full((1, 1), m + 0.25 * m, jnp.float32)
        p = hist_acc[...] / n_tokens
        ent = jnp.sum(p * jnp.log(p + 1e-10), keepdims=True)
        perp_ref[...] = jnp.exp(-ent).reshape(1, 1)


def _sc_enc_body(idx_hbm, enc_hbm, buf_v, idx_v, *, rw, nch):
    c = lax.axis_index("c")
    s_ = lax.axis_index("s")
    wid = s_ * 2 + c
    base = wid * rw
    pltpu.sync_copy(idx_hbm.at[pl.ds(base, rw)], idx_v)
    zero16 = jnp.zeros((16,), jnp.float32)
    one16 = jnp.ones((16,), jnp.float32)
    lane = lax.iota(jnp.int32, 16)

    def zblk(j, carry):
        for u in range(8):
            buf_v[pl.ds((j * 8 + u) * 16, 16)] = zero16
        return carry
    lax.fori_loop(0, (_CH * _K) // 128, zblk, 0)

    def chunk(ci, carry):
        row0 = ci * _CH
        for r4 in range(_CH // 16):
            idxs = idx_v[pl.ds(row0 + r4 * 16, 16)]
            for u in range(16):
                iv = idxs[u]
                off = (r4 * 16 + u) * _K + (iv // 16) * 16
                vec = jnp.where(lane == (iv % 16), 1.0, 0.0)
                buf_v[pl.ds(off, 16)] = vec
        pltpu.sync_copy(buf_v, enc_hbm.at[pl.ds((base + row0) * _K, _CH * _K)])
        for r4 in range(_CH // 16):
            idxs = idx_v[pl.ds(row0 + r4 * 16, 16)]
            for u in range(16):
                iv = idxs[u]
                off = (r4 * 16 + u) * _K + (iv // 16) * 16
                buf_v[pl.ds(off, 16)] = zero16
        return carry
    lax.fori_loop(0, nch, chunk, 0)


def kernel(inputs, W):
    b, s, d = inputs.shape
    n = b * s
    xt = jnp.transpose(inputs, (0, 2, 1))
    wt = jnp.transpose(W, (1, 0))
    rows_per_block = _TB // s
    nblocks = n // _TB
    loss, qt, perp, idx3 = pl.pallas_call(
        functools.partial(_vq_block, nblocks=nblocks, n_tokens=n),
        grid=(nblocks,),
        in_specs=[
            pl.BlockSpec((rows_per_block, d, s), lambda i: (i, 0, 0)),
            pl.BlockSpec((_D, _K), lambda i: (0, 0)),
        ],
        out_specs=[
            pl.BlockSpec((1, 1), lambda i: (0, 0)),
            pl.BlockSpec((rows_per_block, d, s), lambda i: (i, 0, 0)),
            pl.BlockSpec((1, 1), lambda i: (0, 0)),
            pl.BlockSpec((1, 1, _TB), lambda i: (i, 0, 0)),
        ],
        out_shape=[
            jax.ShapeDtypeStruct((1, 1), jnp.float32),
            jax.ShapeDtypeStruct((b, d, s), jnp.float32),
            jax.ShapeDtypeStruct((1, 1), jnp.float32),
            jax.ShapeDtypeStruct((nblocks, 1, _TB), jnp.int32),
        ],
        scratch_shapes=[
            pltpu.SMEM((1, 1), jnp.float32),
            pltpu.VMEM((1, _K), jnp.float32),
            pltpu.VMEM((1, _K), jnp.float32),
        ],
    )(xt, wt)
    q = jnp.transpose(qt, (0, 2, 1))
    idx_flat = idx3.reshape(n)
    rw = n // _NW
    mesh = plsc.VectorSubcoreMesh(core_axis_name="c", subcore_axis_name="s")
    enc = pl.kernel(
        functools.partial(_sc_enc_body, rw=rw, nch=rw // _CH),
        mesh=mesh,
        out_type=jax.ShapeDtypeStruct((n * _K,), jnp.float32),
        scratch_types=[
            pltpu.VMEM((_CH * _K,), jnp.float32),
            pltpu.VMEM((rw,), jnp.int32),
        ],
    )(idx_flat)
    return (loss[0, 0], q, perp[0, 0], enc.reshape(n, _K))
